# baseline (speedup 1.0000x reference)
import functools

import jax
import jax.numpy as jnp
from jax import lax
from jax.experimental import pallas as pl
from jax.experimental.pallas import tpu as pltpu
from jax.experimental.pallas import tpu_sc as plsc

B = 4096
L = 20
V = 100000
D = 64
V2 = V // 2

NC = 2
NS = 16
NW = NC * NS
BAGS_W = B // NW
CHUNK = 32
NCHUNK = BAGS_W // CHUNK
IDX_CHUNK = CHUNK * L
IDX_ROWS = IDX_CHUNK // 128


def _repack_body(in_ref, o_ref):
    h = pl.program_id(1)
    x = in_ref[0]

    @pl.when(h == 0)
    def _():
        o_ref[:, pl.ds(0, D)] = x

    @pl.when(h == 1)
    def _():
        o_ref[:, pl.ds(D, D)] = x


@functools.partial(jax.jit, static_argnames=("t",))
def _tc_repack(tab3, t):
    bv2 = 5000
    nb = V2 // bv2
    return pl.pallas_call(
        _repack_body,
        grid=(t * nb, 2),
        in_specs=[
            pl.BlockSpec((1, bv2, D), lambda i, h: (i // nb, i % nb + h * nb, 0)),
        ],
        out_specs=pl.BlockSpec((bv2, 2 * D), lambda i, h: (i, 0)),
        out_shape=jax.ShapeDtypeStruct((t * V2, 2 * D), jnp.float32),
    )(tab3)


def _reduce_chunk(idx_v, rows_v, out_v, c, col0):
    def bag_body(i, _):
        rbase = i * L
        o0 = jnp.where(idx_v[pl.ds(rbase, 16)] >= V2, 64, 0)
        o1 = jnp.where(idx_v[pl.ds(rbase + 4, 16)] >= V2, 64, 0)
        offs = [o0[r] for r in range(16)] + [o1[12 + r] for r in range(4)]
        for d in range(4):
            m = rows_v[rbase, pl.ds(offs[0] + d * 16, 16)]
            for r in range(1, L):
                m = jnp.maximum(
                    m, rows_v[rbase + r, pl.ds(offs[r] + d * 16, 16)])
            out_v[c * CHUNK + i, pl.ds(col0 + d * 16, 16)] = m
        return 0
    lax.fori_loop(0, CHUNK, bag_body, 0)


def _do_feature(tab, idx_flat, t, col0, bag_base, idx_v, idxp_v, rows_v,
                out_v, sem):
    def chunk_body(c, _):
        off = pl.multiple_of(t * (B * L) + (bag_base + c * CHUNK) * L, 128)
        pltpu.sync_copy(idx_flat.at[pl.ds(off, IDX_CHUNK)], idx_v)

        def pack_body(j, _):
            sl = pl.ds(j * 16, 16)
            v = idx_v[sl]
            idxp_v[sl] = jnp.where(v >= V2, v - V2, v) + t * V2
            return 0
        lax.fori_loop(0, IDX_CHUNK // 16, pack_body, 0, unroll=4)

        cps = [
            pltpu.async_copy(tab.at[idxp_v.at[pl.ds(j * 128, 128)]],
                             rows_v.at[pl.ds(j * 128, 128)], sem)
            for j in range(IDX_ROWS)
        ]
        for cp in cps:
            cp.wait()
        _reduce_chunk(idx_v, rows_v, out_v, c, col0)
        return 0
    lax.fori_loop(0, NCHUNK, chunk_body, 0)


def _sc_body_lt(tab, idx_flat, val3, idx_v, idxp_v, rows_v, out_v, sem):
    wid = lax.axis_index("s") * NC + lax.axis_index("c")
    bag_base = wid * BAGS_W

    def pair(p, _):
        _do_feature(tab, idx_flat, 2 * p, 0, bag_base,
                    idx_v, idxp_v, rows_v, out_v, sem)
        _do_feature(tab, idx_flat, 2 * p + 1, 64, bag_base,
                    idx_v, idxp_v, rows_v, out_v, sem)
        pltpu.sync_copy(
            out_v, val3.at[p, pl.ds(pl.multiple_of(bag_base, 8), BAGS_W)])
        return 0
    lax.fori_loop(0, 3, pair, 0)


def _sc_body_gt(tab, show_tab, idx_flat, show_ids, val3,
                idx_v, idxp_v, rows_v, out_v, sem):
    wid = lax.axis_index("s") * NC + lax.axis_index("c")
    bag_base = wid * BAGS_W

    def pair(p, _):
        _do_feature(tab, idx_flat, 2 * p, 0, bag_base,
                    idx_v, idxp_v, rows_v, out_v, sem)
        _do_feature(tab, idx_flat, 2 * p + 1, 64, bag_base,
                    idx_v, idxp_v, rows_v, out_v, sem)
        pltpu.sync_copy(
            out_v, val3.at[p, pl.ds(pl.multiple_of(bag_base, 8), BAGS_W)])
        return 0
    lax.fori_loop(0, 2, pair, 0)

    _do_feature(tab, idx_flat, jnp.int32(4), 0, bag_base,
                idx_v, idxp_v, rows_v, out_v, sem)
    pltpu.sync_copy(
        show_ids.at[pl.ds(pl.multiple_of(bag_base, 128), BAGS_W)],
        idx_v.at[pl.ds(0, BAGS_W)])

    def show_pack(j, _):
        sl = pl.ds(j * 16, 16)
        v = idx_v[sl]
        idxp_v[sl] = jnp.where(v >= V2, v - V2, v)
        return 0
    lax.fori_loop(0, BAGS_W // 16, show_pack, 0, unroll=4)
    pltpu.async_copy(show_tab.at[idxp_v.at[pl.ds(0, BAGS_W)]],
                     rows_v.at[pl.ds(0, BAGS_W)], sem).wait()

    def show_body(g, _):
        ho = jnp.where(idx_v[pl.ds(g * 16, 16)] >= V2, 64, 0)
        for r in range(16):
            i = g * 16 + r
            for d in range(4):
                out_v[i, pl.ds(64 + d * 16, 16)] = (
                    rows_v[i, pl.ds(ho[r] + d * 16, 16)])
        return 0
    lax.fori_loop(0, BAGS_W // 16, show_body, 0)
    pltpu.sync_copy(
        out_v, val3.at[2, pl.ds(pl.multiple_of(bag_base, 8), BAGS_W)])


_SC_SCRATCH = [
    pltpu.VMEM((IDX_CHUNK,), jnp.int32),
    pltpu.VMEM((IDX_CHUNK,), jnp.int32),
    pltpu.VMEM((IDX_CHUNK, 128), jnp.float32),
    pltpu.VMEM((BAGS_W, 128), jnp.float32),
    pltpu.SemaphoreType.DMA,
]


def _sc_mesh():
    return plsc.VectorSubcoreMesh(core_axis_name="c", subcore_axis_name="s",
                                  num_cores=NC, num_subcores=NS)


@jax.jit
def _sc_gather_lt(tab, idx_flat):
    return pl.kernel(
        _sc_body_lt,
        out_type=jax.ShapeDtypeStruct((3, B, 128), jnp.float32),
        mesh=_sc_mesh(),
        scratch_types=_SC_SCRATCH,
    )(tab, idx_flat)


@jax.jit
def _sc_gather_gt(tab, show_tab, idx_flat, show_ids):
    return pl.kernel(
        _sc_body_gt,
        out_type=jax.ShapeDtypeStruct((3, B, 128), jnp.float32),
        mesh=_sc_mesh(),
        scratch_types=_SC_SCRATCH,
    )(tab, show_tab, idx_flat, show_ids)


def _mm_body(va_ref, vb_ref, w_ref, bias_ref, o_ref):
    acc = jnp.dot(va_ref[0], w_ref[0], preferred_element_type=jnp.float32)
    for p in range(1, 3):
        acc += jnp.dot(va_ref[p], w_ref[p], preferred_element_type=jnp.float32)
    for p in range(3):
        acc += jnp.dot(vb_ref[p], w_ref[3 + p],
                       preferred_element_type=jnp.float32)
    o_ref[...] = acc + bias_ref[...]


@jax.jit
def _tc_matmul(val3a, val3b, wc, bias):
    bm = 512
    return pl.pallas_call(
        _mm_body,
        grid=(B // bm,),
        in_specs=[
            pl.BlockSpec((3, bm, 128), lambda i: (0, i, 0)),
            pl.BlockSpec((3, bm, 128), lambda i: (0, i, 0)),
            pl.BlockSpec((6, 128, 128), lambda i: (0, 0, 0)),
            pl.BlockSpec((1, 128), lambda i: (0, 0)),
        ],
        out_specs=pl.BlockSpec((bm, 128), lambda i: (i, 0)),
        out_shape=jax.ShapeDtypeStruct((B, 128), jnp.float32),
    )(val3a, val3b, wc, bias)


def kernel(lt_inputs, gt_inputs, show_ids, lt_tables, gt_tables, show_table,
           W, b):
    lt_p = _tc_repack(lt_tables, 6)
    val3a = _sc_gather_lt(lt_p, lt_inputs.reshape(6 * B * L))
    gt_p = _tc_repack(gt_tables, 5)
    show_p = _tc_repack(show_table.reshape(1, V, D), 1)
    val3b = _sc_gather_gt(gt_p, show_p, gt_inputs.reshape(5 * B * L),
                          show_ids)

    wc = W.transpose(1, 0, 2).reshape(12 * D, 25)
    wc = jnp.pad(wc, ((0, 0), (0, 103))).reshape(6, 128, 128)
    bias = jnp.pad(b.reshape(1, 25), ((0, 0), (0, 103)))
    out = _tc_matmul(val3a, val3b, wc, bias)
    return out[:, :25].reshape(B, 5, 5).transpose(1, 0, 2)


# speedup vs baseline: 1.3253x; 1.3253x over previous
import functools

import jax
import jax.numpy as jnp
from jax import lax
from jax.experimental import pallas as pl
from jax.experimental.pallas import tpu as pltpu
from jax.experimental.pallas import tpu_sc as plsc

B = 4096
L = 20
V = 100000
D = 64
H = 50048

NC = 2
NS = 16
NW = NC * NS
BAGS_W = B // NW
CHUNK = 16
NCHUNK = BAGS_W // CHUNK
IDX_CHUNK = CHUNK * L
FIRES = ((0, 128), (128, 128), (256, 64))


_BV = 2176
_NB = H // _BV


def _repack_body(in_ref, o_ref):
    h = pl.program_id(1)
    xt = jnp.transpose(in_ref[0], (1, 0))

    @pl.when(h == 0)
    def _():
        o_ref[:, pl.ds(0, D)] = xt

    @pl.when(h == 1)
    def _():
        o_ref[:, pl.ds(D, D)] = xt


@functools.partial(jax.jit, static_argnames=("t",))
def _tc_repack(tab_t, t):
    return pl.pallas_call(
        _repack_body,
        grid=(t * _NB, 2),
        in_specs=[
            pl.BlockSpec((1, D, _BV),
                         lambda i, h: (i // _NB, 0, i % _NB + h * _NB)),
        ],
        out_specs=pl.BlockSpec((_BV, 2 * D), lambda i, h: (i, 0)),
        out_shape=jax.ShapeDtypeStruct((t * H, 2 * D), jnp.float32),
    )(tab_t)


def _reduce_chunk(idx_v, rows_v, out_v, c, col0, base):
    def bag_body(i, _):
        rbase = base + i * L
        o0 = jnp.where(idx_v[pl.ds(rbase, 16)] >= H, 64, 0)
        o1 = jnp.where(idx_v[pl.ds(rbase + 4, 16)] >= H, 64, 0)
        offs = [o0[r] for r in range(16)] + [o1[12 + r] for r in range(4)]
        for d in range(4):
            m = rows_v[rbase, pl.ds(offs[0] + d * 16, 16)]
            for r in range(1, L):
                m = jnp.maximum(
                    m, rows_v[rbase + r, pl.ds(offs[r] + d * 16, 16)])
            out_v[c * CHUNK + i, pl.ds(col0 + d * 16, 16)] = m
        return 0
    lax.fori_loop(0, CHUNK, bag_body, 0)


def _do_feature(tab, idx_flat, t, col0, bag_base, idx_v, idxp_v, rows_v,
                out_v, sem):
    def fire(c, base):
        off = pl.multiple_of(t * (B * L) + (bag_base + c * CHUNK) * L, 128)
        pltpu.sync_copy(idx_flat.at[pl.ds(off, IDX_CHUNK)],
                        idx_v.at[pl.ds(base, IDX_CHUNK)])

        def pack_body(j, _):
            sl = pl.ds(base + j * 16, 16)
            v = idx_v[sl]
            idxp_v[sl] = jnp.where(v >= H, v - H, v) + t * H
            return 0
        lax.fori_loop(0, IDX_CHUNK // 16, pack_body, 0, unroll=4)
        for s, n in FIRES:
            pltpu.async_copy(tab.at[idxp_v.at[pl.ds(base + s, n)]],
                             rows_v.at[pl.ds(base + s, n)], sem)

    def drain(base):
        pltpu.make_async_copy(
            tab.at[pl.ds(0, IDX_CHUNK)],
            rows_v.at[pl.ds(base, IDX_CHUNK)], sem).wait()

    fire(0, 0)

    def chunk_body(c, _):
        base = (c & 1) * IDX_CHUNK
        pbase = IDX_CHUNK - base
        drain(pbase)
        fire(c, base)
        _reduce_chunk(idx_v, rows_v, out_v, c - 1, col0, pbase)
        return 0
    lax.fori_loop(1, NCHUNK, chunk_body, 0)
    last = NCHUNK - 1
    drain((last & 1) * IDX_CHUNK)
    _reduce_chunk(idx_v, rows_v, out_v, last, col0, (last & 1) * IDX_CHUNK)


def _sc_body_lt(tab, idx_flat, val3, idx_v, idxp_v, rows_v, out_v, sem):
    wid = lax.axis_index("s") * NC + lax.axis_index("c")
    bag_base = wid * BAGS_W

    def pair(p, _):
        _do_feature(tab, idx_flat, 2 * p, 0, bag_base,
                    idx_v, idxp_v, rows_v, out_v, sem)
        _do_feature(tab, idx_flat, 2 * p + 1, 64, bag_base,
                    idx_v, idxp_v, rows_v, out_v, sem)
        pltpu.sync_copy(
            out_v, val3.at[p, pl.ds(pl.multiple_of(bag_base, 8), BAGS_W)])
        return 0
    lax.fori_loop(0, 3, pair, 0)


def _sc_body_gt(tab, show_tab, idx_flat, show_ids, val3,
                idx_v, idxp_v, rows_v, out_v, sem):
    wid = lax.axis_index("s") * NC + lax.axis_index("c")
    bag_base = wid * BAGS_W

    def pair(p, _):
        _do_feature(tab, idx_flat, 2 * p, 0, bag_base,
                    idx_v, idxp_v, rows_v, out_v, sem)
        _do_feature(tab, idx_flat, 2 * p + 1, 64, bag_base,
                    idx_v, idxp_v, rows_v, out_v, sem)
        pltpu.sync_copy(
            out_v, val3.at[p, pl.ds(pl.multiple_of(bag_base, 8), BAGS_W)])
        return 0
    lax.fori_loop(0, 2, pair, 0)

    _do_feature(tab, idx_flat, jnp.int32(4), 0, bag_base,
                idx_v, idxp_v, rows_v, out_v, sem)
    pltpu.sync_copy(
        show_ids.at[pl.ds(pl.multiple_of(bag_base, 128), BAGS_W)],
        idx_v.at[pl.ds(0, BAGS_W)])

    def show_pack(j, _):
        sl = pl.ds(j * 16, 16)
        v = idx_v[sl]
        idxp_v[sl] = jnp.where(v >= H, v - H, v)
        return 0
    lax.fori_loop(0, BAGS_W // 16, show_pack, 0, unroll=4)
    pltpu.async_copy(show_tab.at[idxp_v.at[pl.ds(0, BAGS_W)]],
                     rows_v.at[pl.ds(0, BAGS_W)], sem).wait()

    def show_body(g, _):
        ho = jnp.where(idx_v[pl.ds(g * 16, 16)] >= H, 64, 0)
        for r in range(16):
            i = g * 16 + r
            for d in range(4):
                out_v[i, pl.ds(64 + d * 16, 16)] = (
                    rows_v[i, pl.ds(ho[r] + d * 16, 16)])
        return 0
    lax.fori_loop(0, BAGS_W // 16, show_body, 0)
    pltpu.sync_copy(
        out_v, val3.at[2, pl.ds(pl.multiple_of(bag_base, 8), BAGS_W)])


_SC_SCRATCH = [
    pltpu.VMEM((2 * IDX_CHUNK,), jnp.int32),
    pltpu.VMEM((2 * IDX_CHUNK,), jnp.int32),
    pltpu.VMEM((2 * IDX_CHUNK, 128), jnp.float32),
    pltpu.VMEM((BAGS_W, 128), jnp.float32),
    pltpu.SemaphoreType.DMA,
]


def _sc_mesh():
    return plsc.VectorSubcoreMesh(core_axis_name="c", subcore_axis_name="s",
                                  num_cores=NC, num_subcores=NS)


@jax.jit
def _sc_gather_lt(tab, idx_flat):
    return pl.kernel(
        _sc_body_lt,
        out_type=jax.ShapeDtypeStruct((3, B, 128), jnp.float32),
        mesh=_sc_mesh(),
        scratch_types=_SC_SCRATCH,
    )(tab, idx_flat)


@jax.jit
def _sc_gather_gt(tab, show_tab, idx_flat, show_ids):
    return pl.kernel(
        _sc_body_gt,
        out_type=jax.ShapeDtypeStruct((3, B, 128), jnp.float32),
        mesh=_sc_mesh(),
        scratch_types=_SC_SCRATCH,
    )(tab, show_tab, idx_flat, show_ids)


def _mm_body(va_ref, vb_ref, w_ref, bias_ref, o_ref):
    acc = jnp.dot(va_ref[0], w_ref[0], preferred_element_type=jnp.float32)
    for p in range(1, 3):
        acc += jnp.dot(va_ref[p], w_ref[p], preferred_element_type=jnp.float32)
    for p in range(3):
        acc += jnp.dot(vb_ref[p], w_ref[3 + p],
                       preferred_element_type=jnp.float32)
    o_ref[...] = acc + bias_ref[...]


@jax.jit
def _tc_matmul(val3a, val3b, wc, bias):
    bm = 512
    return pl.pallas_call(
        _mm_body,
        grid=(B // bm,),
        in_specs=[
            pl.BlockSpec((3, bm, 128), lambda i: (0, i, 0)),
            pl.BlockSpec((3, bm, 128), lambda i: (0, i, 0)),
            pl.BlockSpec((6, 128, 128), lambda i: (0, 0, 0)),
            pl.BlockSpec((1, 128), lambda i: (0, 0)),
        ],
        out_specs=pl.BlockSpec((bm, 128), lambda i: (i, 0)),
        out_shape=jax.ShapeDtypeStruct((B, 128), jnp.float32),
    )(val3a, val3b, wc, bias)


def kernel(lt_inputs, gt_inputs, show_ids, lt_tables, gt_tables, show_table,
           W, b):
    lt_p = _tc_repack(lt_tables.transpose(0, 2, 1), 6)
    val3a = _sc_gather_lt(lt_p, lt_inputs.reshape(6 * B * L))
    gt_p = _tc_repack(gt_tables.transpose(0, 2, 1), 5)
    show_p = _tc_repack(show_table.T.reshape(1, D, V), 1)
    val3b = _sc_gather_gt(gt_p, show_p, gt_inputs.reshape(5 * B * L),
                          show_ids)

    wc = W.transpose(1, 0, 2).reshape(12 * D, 25)
    wc = jnp.pad(wc, ((0, 0), (0, 103))).reshape(6, 128, 128)
    bias = jnp.pad(b.reshape(1, 25), ((0, 0), (0, 103)))
    out = _tc_matmul(val3a, val3b, wc, bias)
    return out[:, :25].reshape(B, 5, 5).transpose(1, 0, 2)
